# Initial kernel scaffold; baseline (speedup 1.0000x reference)
import functools

import jax
import jax.numpy as jnp
from jax import lax
from jax.experimental import pallas as pl
from jax.experimental.pallas import tpu as pltpu
from jax.experimental.pallas import tpu_sc as plsc

B = 4096
L = 200
EMBED = 32
HIDDEN = 256
NUM_CLASSES = 10

NC = 2
NS = 16
NW = NC * NS
BPW = B // NW
SUB = 16
NSUB = BPW // SUB
G0, G1 = 128, L - 128

_mesh = plsc.VectorSubcoreMesh(core_axis_name="c", subcore_axis_name="s")


@functools.partial(
    pl.kernel,
    out_type=jax.ShapeDtypeStruct((B, 3 * EMBED), jnp.float32),
    mesh=_mesh,
    scratch_types=[
        pltpu.VMEM((SUB, L), jnp.int32),
        pltpu.VMEM((SUB, L), jnp.int32),
        pltpu.VMEM((SUB, L), jnp.int32),
        pltpu.VMEM((L, EMBED), jnp.float32),
        pltpu.VMEM((L, EMBED), jnp.float32),
        pltpu.VMEM((L, EMBED), jnp.float32),
        pltpu.VMEM((BPW, 3 * EMBED), jnp.float32),
        pltpu.SemaphoreType.DMA,
    ],
)
def _sc_pool(x0_hbm, x2_hbm, x3_hbm, e0_hbm, e2_hbm, e3_hbm, out_hbm,
             idx0, idx2, idx3, r0, r2, r3, obuf, sem):
    wid = lax.axis_index("s") * NC + lax.axis_index("c")
    base = wid * BPW

    for sc in range(NSUB):
        row0 = base + sc * SUB
        pltpu.sync_copy(x0_hbm.at[pl.ds(row0, SUB), :], idx0)
        pltpu.sync_copy(x2_hbm.at[pl.ds(row0, SUB), :], idx2)
        pltpu.sync_copy(x3_hbm.at[pl.ds(row0, SUB), :], idx3)

        def body(bl, carry, sc=sc):
            copies = []
            for e_hbm, idx, r in ((e0_hbm, idx0, r0),
                                  (e2_hbm, idx2, r2),
                                  (e3_hbm, idx3, r3)):
                copies.append(pltpu.async_copy(
                    e_hbm.at[idx.at[bl, pl.ds(0, G0)]], r.at[pl.ds(0, G0)], sem))
                copies.append(pltpu.async_copy(
                    e_hbm.at[idx.at[bl, pl.ds(G0, G1)]], r.at[pl.ds(G0, G1)], sem))
            for c in copies:
                c.wait()

            def acc(j, a):
                return (a[0] + r0[j, pl.ds(0, 16)],
                        a[1] + r0[j, pl.ds(16, 16)],
                        a[2] + r2[j, pl.ds(0, 16)],
                        a[3] + r2[j, pl.ds(16, 16)],
                        a[4] + r3[j, pl.ds(0, 16)],
                        a[5] + r3[j, pl.ds(16, 16)])

            z = jnp.zeros((16,), jnp.float32)
            a = lax.fori_loop(0, L, acc, (z, z, z, z, z, z))
            bo = sc * SUB + bl
            for t in range(6):
                obuf[bo, pl.ds(t * 16, 16)] = a[t]
            return carry

        lax.fori_loop(0, SUB, body, 0)

    pltpu.sync_copy(obuf, out_hbm.at[pl.ds(base, BPW), :])


def _mlp_body(s_ref, w1_ref, b1_ref, w2_ref, b2_ref, o_ref):
    x = s_ref[:] * (1.0 / L)
    h = jnp.dot(x, w1_ref[:], preferred_element_type=jnp.float32,
                precision=lax.Precision.HIGHEST) + b1_ref[:]
    mean = jnp.mean(h, axis=0, keepdims=True)
    c = h - mean
    var = jnp.mean(c * c, axis=0, keepdims=True)
    hn = c * lax.rsqrt(var + 1e-5)
    g = 0.5 * hn * (1.0 + lax.erf(hn * (2.0 ** -0.5)))
    o_ref[:] = jnp.dot(g, w2_ref[:], preferred_element_type=jnp.float32,
                       precision=lax.Precision.HIGHEST) + b2_ref[:]


_mlp = pl.pallas_call(
    _mlp_body,
    out_shape=jax.ShapeDtypeStruct((B, NUM_CLASSES), jnp.float32),
)


def kernel(x0, x1, x2, x3, emb_word, emb_ngram2, emb_ngram3, W1, b1, W2, b2):
    del x1
    x0 = x0.astype(jnp.int32)
    x2 = x2.astype(jnp.int32)
    x3 = x3.astype(jnp.int32)
    pooled = _sc_pool(x0, x2, x3, emb_word, emb_ngram2, emb_ngram3)
    return _mlp(pooled, W1, b1.reshape(1, HIDDEN), W2, b2.reshape(1, NUM_CLASSES))


# speedup vs baseline: 4.6158x; 4.6158x over previous
import functools

import jax
import jax.numpy as jnp
from jax import lax
from jax.experimental import pallas as pl
from jax.experimental.pallas import tpu as pltpu
from jax.experimental.pallas import tpu_sc as plsc

B = 4096
L = 200
EMBED = 32
HIDDEN = 256
NUM_CLASSES = 10

NC = 2
NS = 16
NW = NC * NS
BPW = B // NW
SUB = 16
NSUB = BPW // SUB
G0, G1 = 128, L - 128

_mesh = plsc.VectorSubcoreMesh(core_axis_name="c", subcore_axis_name="s")


@functools.partial(
    pl.kernel,
    out_type=jax.ShapeDtypeStruct((B, 3 * EMBED), jnp.float32),
    mesh=_mesh,
    scratch_types=[
        pltpu.VMEM((SUB, L), jnp.int32),
        pltpu.VMEM((SUB, L), jnp.int32),
        pltpu.VMEM((SUB, L), jnp.int32),
        pltpu.VMEM((L, EMBED), jnp.float32),
        pltpu.VMEM((L, EMBED), jnp.float32),
        pltpu.VMEM((L, EMBED), jnp.float32),
        pltpu.VMEM((BPW, 3 * EMBED), jnp.float32),
        pltpu.SemaphoreType.DMA,
    ],
    compiler_params=pltpu.CompilerParams(use_tc_tiling_on_sc=False),
)
def _sc_pool(x0_hbm, x2_hbm, x3_hbm, e0_hbm, e2_hbm, e3_hbm, out_hbm,
             idx0, idx2, idx3, r0, r2, r3, obuf, sem):
    wid = lax.axis_index("s") * NC + lax.axis_index("c")
    base = wid * BPW

    for sc in range(NSUB):
        row0 = base + sc * SUB
        pltpu.sync_copy(x0_hbm.at[pl.ds(row0, SUB), :], idx0)
        pltpu.sync_copy(x2_hbm.at[pl.ds(row0, SUB), :], idx2)
        pltpu.sync_copy(x3_hbm.at[pl.ds(row0, SUB), :], idx3)

        def body(bl, carry, sc=sc):
            copies = []
            for e_hbm, idx, r in ((e0_hbm, idx0, r0),
                                  (e2_hbm, idx2, r2),
                                  (e3_hbm, idx3, r3)):
                copies.append(pltpu.async_copy(
                    e_hbm.at[idx.at[bl, pl.ds(0, G0)]], r.at[pl.ds(0, G0)], sem))
                copies.append(pltpu.async_copy(
                    e_hbm.at[idx.at[bl, pl.ds(G0, G1)]], r.at[pl.ds(G0, G1)], sem))
            for c in copies:
                c.wait()

            def acc(j, a):
                return (a[0] + r0[j, pl.ds(0, 16)],
                        a[1] + r0[j, pl.ds(16, 16)],
                        a[2] + r2[j, pl.ds(0, 16)],
                        a[3] + r2[j, pl.ds(16, 16)],
                        a[4] + r3[j, pl.ds(0, 16)],
                        a[5] + r3[j, pl.ds(16, 16)])

            z = jnp.zeros((16,), jnp.float32)
            a = lax.fori_loop(0, L, acc, (z, z, z, z, z, z))
            bo = sc * SUB + bl
            for t in range(6):
                obuf[bo, pl.ds(t * 16, 16)] = a[t]
            return carry

        lax.fori_loop(0, SUB, body, 0)

    pltpu.sync_copy(obuf, out_hbm.at[pl.ds(base, BPW), :])


def _mlp_body(s_ref, w1_ref, b1_ref, w2_ref, b2_ref, o_ref):
    x = s_ref[:] * (1.0 / L)
    h = jnp.dot(x, w1_ref[:], preferred_element_type=jnp.float32,
                precision=lax.Precision.HIGHEST) + b1_ref[:]
    mean = jnp.mean(h, axis=0, keepdims=True)
    c = h - mean
    var = jnp.mean(c * c, axis=0, keepdims=True)
    hn = c * lax.rsqrt(var + 1e-5)
    g = 0.5 * hn * (1.0 + lax.erf(hn * (2.0 ** -0.5)))
    o_ref[:] = jnp.dot(g, w2_ref[:], preferred_element_type=jnp.float32,
                       precision=lax.Precision.HIGHEST) + b2_ref[:]


_mlp = pl.pallas_call(
    _mlp_body,
    out_shape=jax.ShapeDtypeStruct((B, NUM_CLASSES), jnp.float32),
)


def kernel(x0, x1, x2, x3, emb_word, emb_ngram2, emb_ngram3, W1, b1, W2, b2):
    del x1
    x0 = x0.astype(jnp.int32)
    x2 = x2.astype(jnp.int32)
    x3 = x3.astype(jnp.int32)
    pooled = _sc_pool(x0, x2, x3, emb_word, emb_ngram2, emb_ngram3)
    return _mlp(pooled, W1, b1.reshape(1, HIDDEN), W2, b2.reshape(1, NUM_CLASSES))


# speedup vs baseline: 5.1144x; 1.1080x over previous
import functools

import jax
import jax.numpy as jnp
from jax import lax
from jax.experimental import pallas as pl
from jax.experimental.pallas import tpu as pltpu
from jax.experimental.pallas import tpu_sc as plsc

B = 4096
L = 200
EMBED = 32
HIDDEN = 256
NUM_CLASSES = 10

NC = 2
NS = 16
NW = NC * NS
BPW = B // NW
SUB = 16
NSUB = BPW // SUB
G0, G1 = 128, L - 128

_mesh = plsc.VectorSubcoreMesh(core_axis_name="c", subcore_axis_name="s")


@functools.partial(
    pl.kernel,
    out_type=jax.ShapeDtypeStruct((B, 3 * EMBED), jnp.float32),
    mesh=_mesh,
    scratch_types=[
        pltpu.VMEM((SUB, L), jnp.int32),
        pltpu.VMEM((SUB, L), jnp.int32),
        pltpu.VMEM((SUB, L), jnp.int32),
        [pltpu.VMEM((L, EMBED), jnp.float32)] * 3,
        [pltpu.VMEM((L, EMBED), jnp.float32)] * 3,
        pltpu.VMEM((SUB, 3 * EMBED), jnp.float32),
        pltpu.SemaphoreType.DMA,
        pltpu.SemaphoreType.DMA,
    ],
    compiler_params=pltpu.CompilerParams(use_tc_tiling_on_sc=False),
)
def _sc_pool(x0_hbm, x2_hbm, x3_hbm, e0_hbm, e2_hbm, e3_hbm, out_hbm,
             idx0, idx2, idx3, bank0, bank1, obuf, semA, semB):
    wid = lax.axis_index("s") * NC + lax.axis_index("c")
    base = wid * BPW
    tables = (e0_hbm, e2_hbm, e3_hbm)
    idxs = (idx0, idx2, idx3)

    def fire(bl, bank, sem):
        handles = []
        for e_hbm, idx, r in zip(tables, idxs, bank):
            handles.append(pltpu.async_copy(
                e_hbm.at[idx.at[bl, pl.ds(0, G0)]], r.at[pl.ds(0, G0)], sem))
            handles.append(pltpu.async_copy(
                e_hbm.at[idx.at[bl, pl.ds(G0, G1)]], r.at[pl.ds(G0, G1)], sem))
        return handles

    def wait_bank(bank, sem):
        for e_hbm, idx, r in zip(tables, idxs, bank):
            pltpu.make_async_copy(
                e_hbm.at[idx.at[0, pl.ds(0, G0)]], r.at[pl.ds(0, G0)], sem).wait()
            pltpu.make_async_copy(
                e_hbm.at[idx.at[0, pl.ds(G0, G1)]], r.at[pl.ds(G0, G1)], sem).wait()

    def acc_bank(bl, bank):
        r0, r2, r3 = bank

        def acc(j, a):
            return (a[0] + r0[j, pl.ds(0, 16)],
                    a[1] + r0[j, pl.ds(16, 16)],
                    a[2] + r2[j, pl.ds(0, 16)],
                    a[3] + r2[j, pl.ds(16, 16)],
                    a[4] + r3[j, pl.ds(0, 16)],
                    a[5] + r3[j, pl.ds(16, 16)])

        z = jnp.zeros((16,), jnp.float32)
        a = lax.fori_loop(0, L, acc, (z, z, z, z, z, z))
        for t in range(6):
            obuf[bl, pl.ds(t * 16, 16)] = a[t]

    for sc in range(NSUB):
        row0 = base + sc * SUB
        pltpu.sync_copy(x0_hbm.at[pl.ds(row0, SUB), :], idx0)
        pltpu.sync_copy(x2_hbm.at[pl.ds(row0, SUB), :], idx2)
        pltpu.sync_copy(x3_hbm.at[pl.ds(row0, SUB), :], idx3)

        fire(0, bank0, semA)

        def pair(g, carry):
            h1 = fire(2 * g + 1, bank1, semB)
            wait_bank(bank0, semA)
            acc_bank(2 * g, bank0)

            @pl.when(g < SUB // 2 - 1)
            def _next():
                fire(2 * g + 2, bank0, semA)

            for h in h1:
                h.wait()
            acc_bank(2 * g + 1, bank1)
            return carry

        lax.fori_loop(0, SUB // 2, pair, 0)
        pltpu.sync_copy(obuf, out_hbm.at[pl.ds(row0, SUB), :])


def _mlp_body(s_ref, w1_ref, b1_ref, w2_ref, b2_ref, o_ref):
    x = s_ref[:] * (1.0 / L)
    h = jnp.dot(x, w1_ref[:], preferred_element_type=jnp.float32,
                precision=lax.Precision.HIGHEST) + b1_ref[:]
    mean = jnp.mean(h, axis=0, keepdims=True)
    c = h - mean
    var = jnp.mean(c * c, axis=0, keepdims=True)
    hn = c * lax.rsqrt(var + 1e-5)
    g = 0.5 * hn * (1.0 + lax.erf(hn * (2.0 ** -0.5)))
    o_ref[:] = jnp.dot(g, w2_ref[:], preferred_element_type=jnp.float32,
                       precision=lax.Precision.HIGHEST) + b2_ref[:]


_mlp = pl.pallas_call(
    _mlp_body,
    out_shape=jax.ShapeDtypeStruct((B, NUM_CLASSES), jnp.float32),
)


def kernel(x0, x1, x2, x3, emb_word, emb_ngram2, emb_ngram3, W1, b1, W2, b2):
    del x1
    x0 = x0.astype(jnp.int32)
    x2 = x2.astype(jnp.int32)
    x3 = x3.astype(jnp.int32)
    pooled = _sc_pool(x0, x2, x3, emb_word, emb_ngram2, emb_ngram3)
    return _mlp(pooled, W1, b1.reshape(1, HIDDEN), W2, b2.reshape(1, NUM_CLASSES))


# speedup vs baseline: 5.2922x; 1.0348x over previous
import functools

import jax
import jax.numpy as jnp
from jax import lax
from jax.experimental import pallas as pl
from jax.experimental.pallas import tpu as pltpu
from jax.experimental.pallas import tpu_sc as plsc

B = 4096
L = 200
EMBED = 32
HIDDEN = 256
NUM_CLASSES = 10

NC = 2
NS = 16
NW = NC * NS
BPW = B // NW
SUB = 16
NSUB = BPW // SUB
G0, G1 = 128, L - 128

_mesh = plsc.VectorSubcoreMesh(core_axis_name="c", subcore_axis_name="s")


@functools.partial(
    pl.kernel,
    out_type=jax.ShapeDtypeStruct((B, EMBED), jnp.float32),
    mesh=_mesh,
    scratch_types=[
        pltpu.VMEM((SUB, L), jnp.int32),
        pltpu.VMEM((L, EMBED), jnp.float32),
        pltpu.VMEM((L, EMBED), jnp.float32),
        pltpu.VMEM((SUB, EMBED), jnp.float32),
        pltpu.SemaphoreType.DMA,
        pltpu.SemaphoreType.DMA,
    ],
    compiler_params=pltpu.CompilerParams(use_tc_tiling_on_sc=False),
)
def _sc_pool(x_hbm, e_hbm, out_hbm, idx, bank0, bank1, obuf, semA, semB):
    wid = lax.axis_index("s") * NC + lax.axis_index("c")
    base = wid * BPW

    def fire(bl, r, sem):
        return (
            pltpu.async_copy(
                e_hbm.at[idx.at[bl, pl.ds(0, G0)]], r.at[pl.ds(0, G0)], sem),
            pltpu.async_copy(
                e_hbm.at[idx.at[bl, pl.ds(G0, G1)]], r.at[pl.ds(G0, G1)], sem),
        )

    def wait_bank(r, sem):
        pltpu.make_async_copy(
            e_hbm.at[idx.at[0, pl.ds(0, G0)]], r.at[pl.ds(0, G0)], sem).wait()
        pltpu.make_async_copy(
            e_hbm.at[idx.at[0, pl.ds(G0, G1)]], r.at[pl.ds(G0, G1)], sem).wait()

    def acc_bank(bl, r):
        def acc(j, a):
            return (a[0] + r[j, pl.ds(0, 16)],
                    a[1] + r[j, pl.ds(16, 16)])

        z = jnp.zeros((16,), jnp.float32)
        a = lax.fori_loop(0, L, acc, (z, z))
        obuf[bl, pl.ds(0, 16)] = a[0]
        obuf[bl, pl.ds(16, 16)] = a[1]

    for sc in range(NSUB):
        row0 = base + sc * SUB
        pltpu.sync_copy(x_hbm.at[pl.ds(row0, SUB), :], idx)
        fire(0, bank0, semA)

        def pair(g, carry):
            h1, h2 = fire(2 * g + 1, bank1, semB)
            wait_bank(bank0, semA)
            acc_bank(2 * g, bank0)

            @pl.when(g < SUB // 2 - 1)
            def _next():
                fire(2 * g + 2, bank0, semA)

            h1.wait()
            h2.wait()
            acc_bank(2 * g + 1, bank1)
            return carry

        lax.fori_loop(0, SUB // 2, pair, 0)
        pltpu.sync_copy(obuf, out_hbm.at[pl.ds(row0, SUB), :])


def _mlp_body(p0_ref, p2_ref, p3_ref, w1_ref, b1_ref, w2_ref, b2_ref, o_ref):
    x = jnp.concatenate([p0_ref[:], p2_ref[:], p3_ref[:]], axis=1) * (1.0 / L)
    h = jnp.dot(x, w1_ref[:], preferred_element_type=jnp.float32,
                precision=lax.Precision.HIGHEST) + b1_ref[:]
    mean = jnp.mean(h, axis=0, keepdims=True)
    c = h - mean
    var = jnp.mean(c * c, axis=0, keepdims=True)
    hn = c * lax.rsqrt(var + 1e-5)
    g = 0.5 * hn * (1.0 + lax.erf(hn * (2.0 ** -0.5)))
    o_ref[:] = jnp.dot(g, w2_ref[:], preferred_element_type=jnp.float32,
                       precision=lax.Precision.HIGHEST) + b2_ref[:]


_mlp = pl.pallas_call(
    _mlp_body,
    out_shape=jax.ShapeDtypeStruct((B, NUM_CLASSES), jnp.float32),
)


def kernel(x0, x1, x2, x3, emb_word, emb_ngram2, emb_ngram3, W1, b1, W2, b2):
    del x1
    x0 = x0.astype(jnp.int32)
    x2 = x2.astype(jnp.int32)
    x3 = x3.astype(jnp.int32)
    p0 = _sc_pool(x0, emb_word)
    p2 = _sc_pool(x2, emb_ngram2)
    p3 = _sc_pool(x3, emb_ngram3)
    return _mlp(p0, p2, p3, W1, b1.reshape(1, HIDDEN),
                W2, b2.reshape(1, NUM_CLASSES))
